# baseline (speedup 1.0000x reference)
import functools

import jax
import jax.numpy as jnp
from jax import lax
from jax.experimental import pallas as pl
from jax.experimental.pallas import tpu as pltpu
from jax.experimental.pallas import tpu_sc as plsc

_L = 16
_BT = 128
_NBUF = 3


def _sc_time_encode(xt, tt, tabf, H):
    L, H_, B = xt.shape
    NW = 32
    assert B == NW * _BT and H_ == H and L % 8 == 0
    NG = _BT // _L
    mesh = plsc.VectorSubcoreMesh(core_axis_name="c", subcore_axis_name="s")

    @functools.partial(
        pl.kernel,
        out_type=jax.ShapeDtypeStruct((L, H, B), jnp.float32),
        mesh=mesh,
        compiler_params=pltpu.CompilerParams(use_tc_tiling_on_sc=True,
                                             needs_layout_passes=False),
        scratch_types=[
            pltpu.VMEM((tabf.shape[0],), jnp.float32),
            pltpu.VMEM((8, _BT), jnp.int32),
            pltpu.VMEM((_NBUF, H, _BT), jnp.float32),
            pltpu.SemaphoreType.DMA((_NBUF,)),
            pltpu.SemaphoreType.DMA((_NBUF,)),
        ],
    )
    def k(x_hbm, t_hbm, tab_hbm, out_hbm, tab_v, tv, buf_v, sx, so):
        wid = lax.axis_index("s") * 2 + lax.axis_index("c")
        b0 = pl.multiple_of(wid * _BT, _BT)
        pltpu.sync_copy(tab_hbm, tab_v)

        def in_copy(l, b):
            return pltpu.make_async_copy(x_hbm.at[l, :, pl.ds(b0, _BT)],
                                         buf_v.at[b], sx.at[b])

        def out_copy(l, b):
            return pltpu.make_async_copy(buf_v.at[b],
                                         out_hbm.at[l, :, pl.ds(b0, _BT)],
                                         so.at[b])

        for l in range(_NBUF - 1):
            in_copy(l, l).start()

        def step(l, _):
            b = lax.rem(l, _NBUF)

            @pl.when(lax.rem(l, 8) == 0)
            def _():
                l8 = pl.multiple_of(l, 8)
                pltpu.sync_copy(t_hbm.at[pl.ds(l8, 8), pl.ds(b0, _BT)], tv)

            in_copy(l, b).wait()

            li = lax.rem(l, 8)
            t64s = [tv[li, pl.ds(g * _L, _L)] * H for g in range(NG)]

            @plsc.parallel_loop(0, H, 1, unroll=2)
            def hbody(h):
                for g in range(NG):
                    val = plsc.load_gather(tab_v, [t64s[g] + h])
                    plsc.addupdate(buf_v.at[b, h, pl.ds(g * _L, _L)], val)

            out_copy(l, b).start()

            lp = l + _NBUF - 1

            @pl.when(lp < L)
            def _():
                b2 = lax.rem(lp, _NBUF)

                @pl.when(l >= 1)
                def _():
                    out_copy(l - 1, b2).wait()

                in_copy(lp, b2).start()

            return ()

        lax.fori_loop(0, L, step, ())

        for l in range(L - _NBUF, L):
            out_copy(l, l % _NBUF).wait()

    return k(xt, tt, tabf)


def kernel(inputs, times, table):
    B, L, H = inputs.shape
    NP = table.shape[0]

    TROWS = 32
    tabf = jnp.zeros((TROWS, H), jnp.float32).at[:NP].set(table).reshape(-1)
    t2 = times.astype(jnp.int32).at[:, 0].set(TROWS - 1)

    xt = jnp.transpose(inputs, (1, 2, 0))
    tt = jnp.transpose(t2, (1, 0))

    out_t = _sc_time_encode(xt, tt, tabf, H)
    return jnp.transpose(out_t, (2, 0, 1))
